# baseline (speedup 1.0000x reference)
import functools

import jax
import jax.numpy as jnp
from jax import lax
from jax.experimental import pallas as pl
from jax.experimental.pallas import tpu as pltpu
from jax.experimental.pallas import tpu_sc as plsc

N_NODES = 10000
N_PAD = 10112
E_EDGES = 320000
D = 128
NUM_CORES = 2
NUM_SUBCORES = 16
NW = NUM_CORES * NUM_SUBCORES
CHUNK = 128
NCHUNKS = 80
EPW = NCHUNKS * CHUNK
E_PAD = EPW * NW
ROWS_PER_TILE = N_PAD // NUM_SUBCORES

@functools.cache
def _get_degree_kernel():
    mesh = plsc.VectorSubcoreMesh(
        core_axis_name="c", subcore_axis_name="s",
        num_cores=NUM_CORES, num_subcores=NUM_SUBCORES)
    return functools.partial(
        pl.kernel,
        out_type=jax.ShapeDtypeStruct((NUM_CORES, N_PAD, D), jnp.float32),
        mesh=mesh,
        scratch_types=[
            pltpu.VMEM((NCHUNKS // 2, CHUNK), jnp.int32),
            pltpu.VMEM((NCHUNKS // 2, CHUNK), jnp.int32),
            pltpu.VMEM((CHUNK, D), jnp.float32),
            pltpu.VMEM((CHUNK, D), jnp.float32),
            pltpu.VMEM_SHARED((N_PAD, D), jnp.float32),
        ],
    )(_degree_kernel_body)


def _degree_kernel_body(src_hbm, dst_hbm, ones_s_hbm, ones_d_hbm, zeros_hbm,
                        degp_hbm, sidx2d, didx2d, ones_s, ones_d, acc):
    cid = lax.axis_index("c")
    sid = lax.axis_index("s")
    wid = sid * NUM_CORES + cid
    rows_lo = sid * ROWS_PER_TILE

    pltpu.sync_copy(zeros_hbm.at[pl.ds(rows_lo, ROWS_PER_TILE)],
                    acc.at[pl.ds(rows_lo, ROWS_PER_TILE)])
    pltpu.sync_copy(ones_s_hbm, ones_s)
    pltpu.sync_copy(ones_d_hbm, ones_d)
    plsc.subcore_barrier()

    for h in range(2):
        pltpu.sync_copy(src_hbm.at[wid, pl.ds(h * _HALF, _HALF)], sidx2d)
        pltpu.sync_copy(dst_hbm.at[wid, pl.ds(h * _HALF, _HALF)], didx2d)

        def body(c, carry):
            pltpu.sync_copy(ones_s, acc.at[sidx2d.at[c]], add=True)
            pltpu.sync_copy(ones_d, acc.at[didx2d.at[c]], add=True)
            return carry

        lax.fori_loop(0, _HALF, body, 0)

    plsc.subcore_barrier()

    pltpu.sync_copy(acc.at[pl.ds(rows_lo, ROWS_PER_TILE)],
                    degp_hbm.at[cid, pl.ds(rows_lo, ROWS_PER_TILE)])


@functools.cache
def _get_agg_kernel():
    mesh = plsc.VectorSubcoreMesh(
        core_axis_name="c", subcore_axis_name="s",
        num_cores=NUM_CORES, num_subcores=NUM_SUBCORES)
    return functools.partial(
        pl.kernel,
        out_type=jax.ShapeDtypeStruct((NUM_CORES, N_PAD, D), jnp.float32),
        mesh=mesh,
        scratch_types=[
            pltpu.VMEM((NCHUNKS // 2, CHUNK), jnp.int32),
            pltpu.VMEM((NCHUNKS // 2, CHUNK), jnp.int32),
            pltpu.VMEM((CHUNK, D), jnp.float32),
            pltpu.VMEM((CHUNK, D), jnp.float32),
            pltpu.VMEM_SHARED((N_PAD, D), jnp.float32),
            pltpu.SemaphoreType.DMA,
            pltpu.SemaphoreType.DMA,
        ],
    )(_agg_kernel_body)


_HALF = NCHUNKS // 2


def _agg_kernel_body(xw_hbm, src_hbm, dst_hbm, zero_hbm, out_hbm,
                     sidx2d, didx2d, rows0, rows1, acc_sh, sem0, sem1):
    cid = lax.axis_index("c")
    sid = lax.axis_index("s")
    wid = sid * NUM_CORES + cid
    rows_lo = sid * ROWS_PER_TILE
    rows_bufs = (rows0, rows1)
    sems = (sem0, sem1)

    pltpu.sync_copy(zero_hbm.at[pl.ds(rows_lo, ROWS_PER_TILE)],
                    acc_sh.at[pl.ds(rows_lo, ROWS_PER_TILE)])
    plsc.subcore_barrier()

    for h in range(2):
        pltpu.sync_copy(src_hbm.at[wid, pl.ds(h * _HALF, _HALF)], sidx2d)
        pltpu.sync_copy(dst_hbm.at[wid, pl.ds(h * _HALF, _HALF)], didx2d)
        pltpu.async_copy(xw_hbm.at[sidx2d.at[0]], rows0, sem0)

        def body(g, carry):
            for b in range(2):
                c = g * 2 + b
                pltpu.make_async_copy(xw_hbm.at[sidx2d.at[c]],
                                      rows_bufs[b], sems[b]).wait()

                @pl.when(c + 1 < _HALF)
                def _():
                    pltpu.async_copy(xw_hbm.at[sidx2d.at[c + 1]],
                                     rows_bufs[1 - b], sems[1 - b])

                pltpu.sync_copy(rows_bufs[b], acc_sh.at[didx2d.at[c]],
                                add=True)
            return carry

        lax.fori_loop(0, _HALF // 2, body, 0)

    plsc.subcore_barrier()
    pltpu.sync_copy(acc_sh.at[pl.ds(rows_lo, ROWS_PER_TILE)],
                    out_hbm.at[cid, pl.ds(rows_lo, ROWS_PER_TILE)])


def _matmul1_body(feats_ref, w1_ref, out_ref):
    out_ref[...] = jnp.dot(feats_ref[...], w1_ref[...],
                           precision=lax.Precision.HIGHEST,
                           preferred_element_type=jnp.float32)


def _norm_scale_body(degp_ref, xwr_ref, xw_ref, norm_ref):
    deg_d = degp_ref[0, :, 0:1] + degp_ref[1, :, 0:1]
    deg_s = degp_ref[0, :, 64:65] + degp_ref[1, :, 64:65]
    ns = lax.rsqrt(jnp.maximum(deg_s, 1.0))
    nd = lax.rsqrt(jnp.maximum(deg_d, 1.0))
    norm_ref[...] = jnp.concatenate([ns, nd], axis=1)
    xw_ref[...] = xwr_ref[...] * ns


def _mid_layer_body(aggp_ref, norm_ref, b1_ref, w2_ref, h1_ref, xw2_ref):
    agg = aggp_ref[0] + aggp_ref[1]
    norm = norm_ref[...]
    ns = norm[:, 0:1]
    nd = norm[:, 1:2]
    h1 = jnp.maximum(agg * nd + b1_ref[...], 0.0)
    h1_ref[...] = h1
    xw2_ref[...] = jnp.dot(h1 * ns, w2_ref[...],
                           precision=lax.Precision.HIGHEST,
                           preferred_element_type=jnp.float32)


def _final_body(aggp_ref, norm_ref, b2_ref, h2_ref):
    agg = aggp_ref[0] + aggp_ref[1]
    nd = norm_ref[...][:, 1:2]
    h2_ref[...] = agg * nd + b2_ref[...]


_matmul1 = pl.pallas_call(
    _matmul1_body,
    out_shape=jax.ShapeDtypeStruct((N_PAD, D), jnp.float32),
)

_norm_scale = pl.pallas_call(
    _norm_scale_body,
    out_shape=(jax.ShapeDtypeStruct((N_PAD, D), jnp.float32),
               jax.ShapeDtypeStruct((N_PAD, 2), jnp.float32)),
)

_mid_layer = pl.pallas_call(
    _mid_layer_body,
    out_shape=(jax.ShapeDtypeStruct((N_PAD, D), jnp.float32),
               jax.ShapeDtypeStruct((N_PAD, D), jnp.float32)),
)

_final_layer = pl.pallas_call(
    _final_body,
    out_shape=jax.ShapeDtypeStruct((N_PAD, D), jnp.float32),
)


def kernel(feats, edge_index, W1, b1, W2, b2):
    src = edge_index[0].astype(jnp.int32)
    dst = edge_index[1].astype(jnp.int32)
    pad = E_PAD - E_EDGES
    dummy = jnp.full((pad,), N_NODES, jnp.int32)
    srcp = jnp.concatenate([src, dummy]).reshape(NW, NCHUNKS, CHUNK)
    dstp = jnp.concatenate([dst, dummy]).reshape(NW, NCHUNKS, CHUNK)
    featsp = jnp.concatenate(
        [feats, jnp.zeros((N_PAD - N_NODES, D), jnp.float32)], axis=0)
    zero_rows = jnp.zeros((N_PAD, D), jnp.float32)
    lane = jnp.arange(D, dtype=jnp.int32)[None, :]
    ones_s = jnp.broadcast_to((lane >= 64).astype(jnp.float32), (CHUNK, D))
    ones_d = jnp.broadcast_to((lane < 64).astype(jnp.float32), (CHUNK, D))

    degree_kernel = _get_degree_kernel()
    agg_kernel = _get_agg_kernel()
    xw1_raw = _matmul1(featsp, W1)
    degp = degree_kernel(srcp, dstp, ones_s, ones_d, zero_rows)
    xw1, norm = _norm_scale(degp, xw1_raw)
    agg1p = agg_kernel(xw1, srcp, dstp, zero_rows)
    h1, xw2 = _mid_layer(agg1p, norm, b1.reshape(1, D), W2)
    agg2p = agg_kernel(xw2, srcp, dstp, zero_rows)
    h2 = _final_layer(agg2p, norm, b2.reshape(1, D))

    h1o = h1[:N_NODES]
    h2o = h2[:N_NODES]
    return ((h1o, h2o), h2o)


# speedup vs baseline: 1.0726x; 1.0726x over previous
import functools

import jax
import jax.numpy as jnp
from jax import lax
from jax.experimental import pallas as pl
from jax.experimental.pallas import tpu as pltpu
from jax.experimental.pallas import tpu_sc as plsc

N_NODES = 10000
N_PAD = 10112
E_EDGES = 320000
D = 128
NUM_CORES = 2
NUM_SUBCORES = 16
NW = NUM_CORES * NUM_SUBCORES
CHUNK = 128
NCHUNKS = 80
EPW = NCHUNKS * CHUNK
E_PAD = EPW * NW
ROWS_PER_TILE = N_PAD // NUM_SUBCORES
NSLABS = 64
SLAB_CHUNKS = 40

@functools.cache
def _get_degree_kernel():
    mesh = plsc.VectorSubcoreMesh(
        core_axis_name="c", subcore_axis_name="s",
        num_cores=NUM_CORES, num_subcores=NUM_SUBCORES)
    return functools.partial(
        pl.kernel,
        out_type=jax.ShapeDtypeStruct((NUM_CORES, N_PAD, D), jnp.float32),
        mesh=mesh,
        scratch_types=[
            pltpu.VMEM((NCHUNKS // 2, CHUNK), jnp.int32),
            pltpu.VMEM((NCHUNKS // 2, CHUNK), jnp.int32),
            pltpu.VMEM((CHUNK, D), jnp.float32),
            pltpu.VMEM((CHUNK, D), jnp.float32),
            pltpu.VMEM_SHARED((N_PAD, D), jnp.float32),
        ],
    )(_degree_kernel_body)


def _degree_kernel_body(src_hbm, dst_hbm, ones_s_hbm, ones_d_hbm, zeros_hbm,
                        degp_hbm, sidx2d, didx2d, ones_s, ones_d, acc):
    cid = lax.axis_index("c")
    sid = lax.axis_index("s")
    wid = sid * NUM_CORES + cid
    rows_lo = sid * ROWS_PER_TILE

    pltpu.sync_copy(zeros_hbm.at[pl.ds(rows_lo, ROWS_PER_TILE)],
                    acc.at[pl.ds(rows_lo, ROWS_PER_TILE)])
    pltpu.sync_copy(ones_s_hbm, ones_s)
    pltpu.sync_copy(ones_d_hbm, ones_d)
    plsc.subcore_barrier()

    for h in range(2):
        pltpu.sync_copy(src_hbm.at[wid, pl.ds(h * _HALF, _HALF)], sidx2d)
        pltpu.sync_copy(dst_hbm.at[wid, pl.ds(h * _HALF, _HALF)], didx2d)

        def body(c, carry):
            pltpu.sync_copy(ones_s, acc.at[sidx2d.at[c]], add=True)
            pltpu.sync_copy(ones_d, acc.at[didx2d.at[c]], add=True)
            return carry

        lax.fori_loop(0, _HALF, body, 0)

    plsc.subcore_barrier()

    pltpu.sync_copy(acc.at[pl.ds(rows_lo, ROWS_PER_TILE)],
                    degp_hbm.at[cid, pl.ds(rows_lo, ROWS_PER_TILE)])


@functools.cache
def _get_agg_kernel():
    mesh = plsc.VectorSubcoreMesh(
        core_axis_name="c", subcore_axis_name="s",
        num_cores=NUM_CORES, num_subcores=NUM_SUBCORES)
    return functools.partial(
        pl.kernel,
        out_type=jax.ShapeDtypeStruct((NUM_CORES, N_PAD, D), jnp.float32),
        mesh=mesh,
        scratch_types=[
            pltpu.VMEM((SLAB_CHUNKS, CHUNK), jnp.int32),
            pltpu.VMEM((SLAB_CHUNKS, CHUNK), jnp.int32),
            pltpu.VMEM((CHUNK, D), jnp.float32),
            pltpu.VMEM((CHUNK, D), jnp.float32),
            pltpu.VMEM_SHARED((N_PAD, D), jnp.float32),
            pltpu.SemaphoreType.DMA,
            pltpu.SemaphoreType.DMA,
        ],
    )(_agg_kernel_body)


_HALF = NCHUNKS // 2


def _agg_kernel_body(xw_hbm, src_hbm, dst_hbm, zero_hbm, out_hbm,
                     sidx2d, didx2d, rows0, rows1, acc_sh, sem0, sem1):
    cid = lax.axis_index("c")
    sid = lax.axis_index("s")
    rows_lo = sid * ROWS_PER_TILE
    rows_bufs = (rows0, rows1)
    sems = (sem0, sem1)

    pltpu.sync_copy(zero_hbm.at[pl.ds(rows_lo, ROWS_PER_TILE)],
                    acc_sh.at[pl.ds(rows_lo, ROWS_PER_TILE)])
    plsc.subcore_barrier()

    def process_slab(slab):
        pltpu.sync_copy(src_hbm.at[slab], sidx2d)
        pltpu.sync_copy(dst_hbm.at[slab], didx2d)
        pltpu.async_copy(xw_hbm.at[sidx2d.at[0]], rows0, sem0)

        def body(g, carry):
            for b in range(2):
                c = g * 2 + b
                pltpu.make_async_copy(xw_hbm.at[sidx2d.at[c]],
                                      rows_bufs[b], sems[b]).wait()

                @pl.when(c + 1 < SLAB_CHUNKS)
                def _():
                    pltpu.async_copy(xw_hbm.at[sidx2d.at[c + 1]],
                                     rows_bufs[1 - b], sems[1 - b])

                pltpu.sync_copy(rows_bufs[b], acc_sh.at[didx2d.at[c]],
                                add=True)
            return carry

        lax.fori_loop(0, SLAB_CHUNKS // 2, body, 0)

    @pl.when(cid == 0)
    def _():
        def sbody(k, carry):
            process_slab(sid * 3 + k)
            return carry

        lax.fori_loop(0, 3, sbody, 0)

    @pl.when(cid == 1)
    def _():
        process_slab(48 + sid)

    plsc.subcore_barrier()
    pltpu.sync_copy(acc_sh.at[pl.ds(rows_lo, ROWS_PER_TILE)],
                    out_hbm.at[cid, pl.ds(rows_lo, ROWS_PER_TILE)])


def _matmul1_body(feats_ref, w1_ref, out_ref):
    out_ref[...] = jnp.dot(feats_ref[...], w1_ref[...],
                           precision=lax.Precision.HIGHEST,
                           preferred_element_type=jnp.float32)


def _norm_scale_body(degp_ref, xwr_ref, xw_ref, norm_ref):
    deg_d = degp_ref[0, :, 0:1] + degp_ref[1, :, 0:1]
    deg_s = degp_ref[0, :, 64:65] + degp_ref[1, :, 64:65]
    ns = lax.rsqrt(jnp.maximum(deg_s, 1.0))
    nd = lax.rsqrt(jnp.maximum(deg_d, 1.0))
    norm_ref[...] = jnp.concatenate([ns, nd], axis=1)
    xw_ref[...] = xwr_ref[...] * ns


def _mid_layer_body(aggp_ref, norm_ref, b1_ref, w2_ref, h1_ref, xw2_ref):
    agg = aggp_ref[0] + aggp_ref[1]
    norm = norm_ref[...]
    ns = norm[:, 0:1]
    nd = norm[:, 1:2]
    h1 = jnp.maximum(agg * nd + b1_ref[...], 0.0)
    h1_ref[...] = h1
    xw2_ref[...] = jnp.dot(h1 * ns, w2_ref[...],
                           precision=lax.Precision.HIGHEST,
                           preferred_element_type=jnp.float32)


def _final_body(aggp_ref, norm_ref, b2_ref, h2_ref):
    agg = aggp_ref[0] + aggp_ref[1]
    nd = norm_ref[...][:, 1:2]
    h2_ref[...] = agg * nd + b2_ref[...]


_matmul1 = pl.pallas_call(
    _matmul1_body,
    out_shape=jax.ShapeDtypeStruct((N_PAD, D), jnp.float32),
)

_norm_scale = pl.pallas_call(
    _norm_scale_body,
    out_shape=(jax.ShapeDtypeStruct((N_PAD, D), jnp.float32),
               jax.ShapeDtypeStruct((N_PAD, 2), jnp.float32)),
)

_mid_layer = pl.pallas_call(
    _mid_layer_body,
    out_shape=(jax.ShapeDtypeStruct((N_PAD, D), jnp.float32),
               jax.ShapeDtypeStruct((N_PAD, D), jnp.float32)),
)

_final_layer = pl.pallas_call(
    _final_body,
    out_shape=jax.ShapeDtypeStruct((N_PAD, D), jnp.float32),
)


def kernel(feats, edge_index, W1, b1, W2, b2):
    src = edge_index[0].astype(jnp.int32)
    dst = edge_index[1].astype(jnp.int32)
    pad = E_PAD - E_EDGES
    dummy = jnp.full((pad,), N_NODES, jnp.int32)
    srcp = jnp.concatenate([src, dummy]).reshape(NW, NCHUNKS, CHUNK)
    dstp = jnp.concatenate([dst, dummy]).reshape(NW, NCHUNKS, CHUNK)
    srcs = srcp.reshape(NSLABS, SLAB_CHUNKS, CHUNK)
    dsts = dstp.reshape(NSLABS, SLAB_CHUNKS, CHUNK)
    featsp = jnp.concatenate(
        [feats, jnp.zeros((N_PAD - N_NODES, D), jnp.float32)], axis=0)
    zero_rows = jnp.zeros((N_PAD, D), jnp.float32)
    lane = jnp.arange(D, dtype=jnp.int32)[None, :]
    ones_s = jnp.broadcast_to((lane >= 64).astype(jnp.float32), (CHUNK, D))
    ones_d = jnp.broadcast_to((lane < 64).astype(jnp.float32), (CHUNK, D))

    degree_kernel = _get_degree_kernel()
    agg_kernel = _get_agg_kernel()
    xw1_raw = _matmul1(featsp, W1)
    degp = degree_kernel(srcp, dstp, ones_s, ones_d, zero_rows)
    xw1, norm = _norm_scale(degp, xw1_raw)
    agg1p = agg_kernel(xw1, srcs, dsts, zero_rows)
    h1, xw2 = _mid_layer(agg1p, norm, b1.reshape(1, D), W2)
    agg2p = agg_kernel(xw2, srcs, dsts, zero_rows)
    h2 = _final_layer(agg2p, norm, b2.reshape(1, D))

    h1o = h1[:N_NODES]
    h2o = h2[:N_NODES]
    return ((h1o, h2o), h2o)


# speedup vs baseline: 1.0773x; 1.0045x over previous
import functools

import jax
import jax.numpy as jnp
from jax import lax
from jax.experimental import pallas as pl
from jax.experimental.pallas import tpu as pltpu
from jax.experimental.pallas import tpu_sc as plsc

N_NODES = 10000
N_PAD = 10112
E_EDGES = 320000
D = 128
NUM_CORES = 2
NUM_SUBCORES = 16
NW = NUM_CORES * NUM_SUBCORES
CHUNK = 128
NCHUNKS = 80
EPW = NCHUNKS * CHUNK
E_PAD = EPW * NW
ROWS_PER_TILE = N_PAD // NUM_SUBCORES
NSLABS = 64
SLAB_CHUNKS = 40

@functools.cache
def _get_degree_kernel():
    mesh = plsc.VectorSubcoreMesh(
        core_axis_name="c", subcore_axis_name="s",
        num_cores=NUM_CORES, num_subcores=NUM_SUBCORES)
    return functools.partial(
        pl.kernel,
        out_type=jax.ShapeDtypeStruct((NUM_CORES, N_PAD, D), jnp.float32),
        mesh=mesh,
        scratch_types=[
            pltpu.VMEM((NCHUNKS // 2, CHUNK), jnp.int32),
            pltpu.VMEM((NCHUNKS // 2, CHUNK), jnp.int32),
            pltpu.VMEM((CHUNK, D), jnp.float32),
            pltpu.VMEM((CHUNK, D), jnp.float32),
            pltpu.VMEM_SHARED((N_PAD, D), jnp.float32),
        ],
    )(_degree_kernel_body)


def _degree_kernel_body(src_hbm, dst_hbm, ones_s_hbm, ones_d_hbm, zeros_hbm,
                        degp_hbm, sidx2d, didx2d, ones_s, ones_d, acc):
    cid = lax.axis_index("c")
    sid = lax.axis_index("s")
    wid = sid * NUM_CORES + cid
    rows_lo = sid * ROWS_PER_TILE

    pltpu.sync_copy(zeros_hbm.at[pl.ds(rows_lo, ROWS_PER_TILE)],
                    acc.at[pl.ds(rows_lo, ROWS_PER_TILE)])
    pltpu.sync_copy(ones_s_hbm, ones_s)
    pltpu.sync_copy(ones_d_hbm, ones_d)
    plsc.subcore_barrier()

    for h in range(2):
        pltpu.sync_copy(src_hbm.at[wid, pl.ds(h * _HALF, _HALF)], sidx2d)
        pltpu.sync_copy(dst_hbm.at[wid, pl.ds(h * _HALF, _HALF)], didx2d)

        def body(c, carry):
            pltpu.sync_copy(ones_s, acc.at[sidx2d.at[c]], add=True)
            pltpu.sync_copy(ones_d, acc.at[didx2d.at[c]], add=True)
            return carry

        lax.fori_loop(0, _HALF, body, 0)

    plsc.subcore_barrier()

    pltpu.sync_copy(acc.at[pl.ds(rows_lo, ROWS_PER_TILE)],
                    degp_hbm.at[cid, pl.ds(rows_lo, ROWS_PER_TILE)])


@functools.cache
def _get_agg_kernel():
    mesh = plsc.VectorSubcoreMesh(
        core_axis_name="c", subcore_axis_name="s",
        num_cores=NUM_CORES, num_subcores=NUM_SUBCORES)
    return functools.partial(
        pl.kernel,
        out_type=jax.ShapeDtypeStruct((NUM_CORES, N_PAD, D), jnp.float32),
        mesh=mesh,
        scratch_types=[
            pltpu.VMEM((SLAB_CHUNKS, CHUNK), jnp.int32),
            pltpu.VMEM((SLAB_CHUNKS, CHUNK), jnp.int32),
            pltpu.VMEM((CHUNK, D), jnp.float32),
            pltpu.VMEM((CHUNK, D), jnp.float32),
            pltpu.VMEM_SHARED((N_PAD, D), jnp.float32),
            pltpu.SemaphoreType.DMA,
            pltpu.SemaphoreType.DMA,
        ],
    )(_agg_kernel_body)


_HALF = NCHUNKS // 2


def _agg_kernel_body(xw_hbm, src_hbm, dst_hbm, zero_hbm, out_hbm,
                     sidx2d, didx2d, rows0, rows1, acc_sh, sem0, sem1):
    cid = lax.axis_index("c")
    sid = lax.axis_index("s")
    rows_lo = sid * ROWS_PER_TILE
    rows_bufs = (rows0, rows1)
    sems = (sem0, sem1)

    pltpu.sync_copy(zero_hbm.at[pl.ds(rows_lo, ROWS_PER_TILE)],
                    acc_sh.at[pl.ds(rows_lo, ROWS_PER_TILE)])
    plsc.subcore_barrier()

    def process_slab(slab):
        pltpu.sync_copy(src_hbm.at[slab], sidx2d)
        pltpu.sync_copy(dst_hbm.at[slab], didx2d)
        pltpu.async_copy(xw_hbm.at[sidx2d.at[0]], rows0, sem0)

        def body(g, carry):
            for b in range(2):
                c = g * 2 + b

                @pl.when(c + 1 < SLAB_CHUNKS)
                def _():
                    pltpu.async_copy(xw_hbm.at[sidx2d.at[c + 1]],
                                     rows_bufs[1 - b], sems[1 - b])

                pltpu.make_async_copy(xw_hbm.at[sidx2d.at[c]],
                                      rows_bufs[b], sems[b]).wait()
                pltpu.sync_copy(rows_bufs[b], acc_sh.at[didx2d.at[c]],
                                add=True)
            return carry

        lax.fori_loop(0, SLAB_CHUNKS // 2, body, 0)

    @pl.when(cid == 0)
    def _():
        def sbody(k, carry):
            process_slab(sid * 3 + k)
            return carry

        lax.fori_loop(0, 3, sbody, 0)

    @pl.when(cid == 1)
    def _():
        process_slab(48 + sid)

    plsc.subcore_barrier()
    pltpu.sync_copy(acc_sh.at[pl.ds(rows_lo, ROWS_PER_TILE)],
                    out_hbm.at[cid, pl.ds(rows_lo, ROWS_PER_TILE)])


def _matmul1_body(feats_ref, w1_ref, out_ref):
    out_ref[...] = jnp.dot(feats_ref[...], w1_ref[...],
                           precision=lax.Precision.HIGHEST,
                           preferred_element_type=jnp.float32)


def _norm_scale_body(degp_ref, xwr_ref, xw_ref, norm_ref):
    deg_d = degp_ref[0, :, 0:1] + degp_ref[1, :, 0:1]
    deg_s = degp_ref[0, :, 64:65] + degp_ref[1, :, 64:65]
    ns = lax.rsqrt(jnp.maximum(deg_s, 1.0))
    nd = lax.rsqrt(jnp.maximum(deg_d, 1.0))
    norm_ref[...] = jnp.concatenate([ns, nd], axis=1)
    xw_ref[...] = xwr_ref[...] * ns


def _mid_layer_body(aggp_ref, norm_ref, b1_ref, w2_ref, h1_ref, xw2_ref):
    agg = aggp_ref[0] + aggp_ref[1]
    norm = norm_ref[...]
    ns = norm[:, 0:1]
    nd = norm[:, 1:2]
    h1 = jnp.maximum(agg * nd + b1_ref[...], 0.0)
    h1_ref[...] = h1
    xw2_ref[...] = jnp.dot(h1 * ns, w2_ref[...],
                           precision=lax.Precision.HIGHEST,
                           preferred_element_type=jnp.float32)


def _final_body(aggp_ref, norm_ref, b2_ref, h2_ref):
    agg = aggp_ref[0] + aggp_ref[1]
    nd = norm_ref[...][:, 1:2]
    h2_ref[...] = agg * nd + b2_ref[...]


_matmul1 = pl.pallas_call(
    _matmul1_body,
    out_shape=jax.ShapeDtypeStruct((N_PAD, D), jnp.float32),
)

_norm_scale = pl.pallas_call(
    _norm_scale_body,
    out_shape=(jax.ShapeDtypeStruct((N_PAD, D), jnp.float32),
               jax.ShapeDtypeStruct((N_PAD, 2), jnp.float32)),
)

_mid_layer = pl.pallas_call(
    _mid_layer_body,
    out_shape=(jax.ShapeDtypeStruct((N_PAD, D), jnp.float32),
               jax.ShapeDtypeStruct((N_PAD, D), jnp.float32)),
)

_final_layer = pl.pallas_call(
    _final_body,
    out_shape=jax.ShapeDtypeStruct((N_PAD, D), jnp.float32),
)


def kernel(feats, edge_index, W1, b1, W2, b2):
    src = edge_index[0].astype(jnp.int32)
    dst = edge_index[1].astype(jnp.int32)
    pad = E_PAD - E_EDGES
    dummy = jnp.full((pad,), N_NODES, jnp.int32)
    srcp = jnp.concatenate([src, dummy]).reshape(NW, NCHUNKS, CHUNK)
    dstp = jnp.concatenate([dst, dummy]).reshape(NW, NCHUNKS, CHUNK)
    srcs = srcp.reshape(NSLABS, SLAB_CHUNKS, CHUNK)
    dsts = dstp.reshape(NSLABS, SLAB_CHUNKS, CHUNK)
    featsp = jnp.concatenate(
        [feats, jnp.zeros((N_PAD - N_NODES, D), jnp.float32)], axis=0)
    zero_rows = jnp.zeros((N_PAD, D), jnp.float32)
    lane = jnp.arange(D, dtype=jnp.int32)[None, :]
    ones_s = jnp.broadcast_to((lane >= 64).astype(jnp.float32), (CHUNK, D))
    ones_d = jnp.broadcast_to((lane < 64).astype(jnp.float32), (CHUNK, D))

    degree_kernel = _get_degree_kernel()
    agg_kernel = _get_agg_kernel()
    xw1_raw = _matmul1(featsp, W1)
    degp = degree_kernel(srcp, dstp, ones_s, ones_d, zero_rows)
    xw1, norm = _norm_scale(degp, xw1_raw)
    agg1p = agg_kernel(xw1, srcs, dsts, zero_rows)
    h1, xw2 = _mid_layer(agg1p, norm, b1.reshape(1, D), W2)
    agg2p = agg_kernel(xw2, srcs, dsts, zero_rows)
    h2 = _final_layer(agg2p, norm, b2.reshape(1, D))

    h1o = h1[:N_NODES]
    h2o = h2[:N_NODES]
    return ((h1o, h2o), h2o)


# speedup vs baseline: 1.0895x; 1.0113x over previous
import functools

import jax
import jax.numpy as jnp
from jax import lax
from jax.experimental import pallas as pl
from jax.experimental.pallas import tpu as pltpu
from jax.experimental.pallas import tpu_sc as plsc

N_NODES = 10000
N_PAD = 10112
E_EDGES = 320000
D = 128
NUM_CORES = 2
NUM_SUBCORES = 16
NW = NUM_CORES * NUM_SUBCORES
CHUNK = 128
NCHUNKS = 80
EPW = NCHUNKS * CHUNK
E_PAD = EPW * NW
ROWS_PER_TILE = N_PAD // NUM_SUBCORES
NSLABS = 128
SLAB_CHUNKS = 20
SLABS_SC0 = 7

@functools.cache
def _get_degree_kernel():
    mesh = plsc.VectorSubcoreMesh(
        core_axis_name="c", subcore_axis_name="s",
        num_cores=NUM_CORES, num_subcores=NUM_SUBCORES)
    return functools.partial(
        pl.kernel,
        out_type=jax.ShapeDtypeStruct((NUM_CORES, N_PAD, D), jnp.float32),
        mesh=mesh,
        scratch_types=[
            pltpu.VMEM((NCHUNKS // 2, CHUNK), jnp.int32),
            pltpu.VMEM((NCHUNKS // 2, CHUNK), jnp.int32),
            pltpu.VMEM((CHUNK, D), jnp.float32),
            pltpu.VMEM((CHUNK, D), jnp.float32),
            pltpu.VMEM_SHARED((N_PAD, D), jnp.float32),
        ],
    )(_degree_kernel_body)


def _degree_kernel_body(src_hbm, dst_hbm, ones_s_hbm, ones_d_hbm, zeros_hbm,
                        degp_hbm, sidx2d, didx2d, ones_s, ones_d, acc):
    cid = lax.axis_index("c")
    sid = lax.axis_index("s")
    wid = sid * NUM_CORES + cid
    rows_lo = sid * ROWS_PER_TILE

    pltpu.sync_copy(zeros_hbm.at[pl.ds(rows_lo, ROWS_PER_TILE)],
                    acc.at[pl.ds(rows_lo, ROWS_PER_TILE)])
    pltpu.sync_copy(ones_s_hbm, ones_s)
    pltpu.sync_copy(ones_d_hbm, ones_d)
    plsc.subcore_barrier()

    for h in range(2):
        pltpu.sync_copy(src_hbm.at[wid, pl.ds(h * _HALF, _HALF)], sidx2d)
        pltpu.sync_copy(dst_hbm.at[wid, pl.ds(h * _HALF, _HALF)], didx2d)

        def body(c, carry):
            pltpu.sync_copy(ones_s, acc.at[sidx2d.at[c]], add=True)
            pltpu.sync_copy(ones_d, acc.at[didx2d.at[c]], add=True)
            return carry

        lax.fori_loop(0, _HALF, body, 0)

    plsc.subcore_barrier()

    pltpu.sync_copy(acc.at[pl.ds(rows_lo, ROWS_PER_TILE)],
                    degp_hbm.at[cid, pl.ds(rows_lo, ROWS_PER_TILE)])


@functools.cache
def _get_agg_kernel():
    mesh = plsc.VectorSubcoreMesh(
        core_axis_name="c", subcore_axis_name="s",
        num_cores=NUM_CORES, num_subcores=NUM_SUBCORES)
    return functools.partial(
        pl.kernel,
        out_type=jax.ShapeDtypeStruct((NUM_CORES, N_PAD, D), jnp.float32),
        mesh=mesh,
        scratch_types=[
            pltpu.VMEM((SLAB_CHUNKS, CHUNK), jnp.int32),
            pltpu.VMEM((SLAB_CHUNKS, CHUNK), jnp.int32),
            pltpu.VMEM((CHUNK, D), jnp.float32),
            pltpu.VMEM((CHUNK, D), jnp.float32),
            pltpu.VMEM_SHARED((N_PAD, D), jnp.float32),
            pltpu.SemaphoreType.DMA,
            pltpu.SemaphoreType.DMA,
        ],
    )(_agg_kernel_body)


_HALF = NCHUNKS // 2


def _agg_kernel_body(xw_hbm, src_hbm, dst_hbm, zero_hbm, out_hbm,
                     sidx2d, didx2d, rows0, rows1, acc_sh, sem0, sem1):
    cid = lax.axis_index("c")
    sid = lax.axis_index("s")
    rows_lo = sid * ROWS_PER_TILE
    rows_bufs = (rows0, rows1)
    sems = (sem0, sem1)

    pltpu.sync_copy(zero_hbm.at[pl.ds(rows_lo, ROWS_PER_TILE)],
                    acc_sh.at[pl.ds(rows_lo, ROWS_PER_TILE)])
    plsc.subcore_barrier()

    def process_slab(slab):
        pltpu.sync_copy(src_hbm.at[slab], sidx2d)
        pltpu.sync_copy(dst_hbm.at[slab], didx2d)
        pltpu.async_copy(xw_hbm.at[sidx2d.at[0]], rows0, sem0)

        def body(g, carry):
            for b in range(2):
                c = g * 2 + b

                @pl.when(c + 1 < SLAB_CHUNKS)
                def _():
                    pltpu.async_copy(xw_hbm.at[sidx2d.at[c + 1]],
                                     rows_bufs[1 - b], sems[1 - b])

                pltpu.make_async_copy(xw_hbm.at[sidx2d.at[c]],
                                      rows_bufs[b], sems[b]).wait()
                pltpu.sync_copy(rows_bufs[b], acc_sh.at[didx2d.at[c]],
                                add=True)
            return carry

        lax.fori_loop(0, SLAB_CHUNKS // 2, body, 0)

    @pl.when(cid == 0)
    def _():
        def sbody(k, carry):
            process_slab(sid * SLABS_SC0 + k)
            return carry

        lax.fori_loop(0, SLABS_SC0, sbody, 0)

    @pl.when(cid == 1)
    def _():
        process_slab(NUM_SUBCORES * SLABS_SC0 + sid)

    plsc.subcore_barrier()
    pltpu.sync_copy(acc_sh.at[pl.ds(rows_lo, ROWS_PER_TILE)],
                    out_hbm.at[cid, pl.ds(rows_lo, ROWS_PER_TILE)])


def _matmul1_body(feats_ref, w1_ref, out_ref):
    out_ref[...] = jnp.dot(feats_ref[...], w1_ref[...],
                           precision=lax.Precision.HIGHEST,
                           preferred_element_type=jnp.float32)


def _norm_scale_body(degp_ref, xwr_ref, xw_ref, norm_ref):
    deg_d = degp_ref[0, :, 0:1] + degp_ref[1, :, 0:1]
    deg_s = degp_ref[0, :, 64:65] + degp_ref[1, :, 64:65]
    ns = lax.rsqrt(jnp.maximum(deg_s, 1.0))
    nd = lax.rsqrt(jnp.maximum(deg_d, 1.0))
    norm_ref[...] = jnp.concatenate([ns, nd], axis=1)
    xw_ref[...] = xwr_ref[...] * ns


def _mid_layer_body(aggp_ref, norm_ref, b1_ref, w2_ref, h1_ref, xw2_ref):
    agg = aggp_ref[0] + aggp_ref[1]
    norm = norm_ref[...]
    ns = norm[:, 0:1]
    nd = norm[:, 1:2]
    h1 = jnp.maximum(agg * nd + b1_ref[...], 0.0)
    h1_ref[...] = h1
    xw2_ref[...] = jnp.dot(h1 * ns, w2_ref[...],
                           precision=lax.Precision.HIGHEST,
                           preferred_element_type=jnp.float32)


def _final_body(aggp_ref, norm_ref, b2_ref, h2_ref):
    agg = aggp_ref[0] + aggp_ref[1]
    nd = norm_ref[...][:, 1:2]
    h2_ref[...] = agg * nd + b2_ref[...]


_matmul1 = pl.pallas_call(
    _matmul1_body,
    out_shape=jax.ShapeDtypeStruct((N_PAD, D), jnp.float32),
)

_norm_scale = pl.pallas_call(
    _norm_scale_body,
    out_shape=(jax.ShapeDtypeStruct((N_PAD, D), jnp.float32),
               jax.ShapeDtypeStruct((N_PAD, 2), jnp.float32)),
)

_mid_layer = pl.pallas_call(
    _mid_layer_body,
    out_shape=(jax.ShapeDtypeStruct((N_PAD, D), jnp.float32),
               jax.ShapeDtypeStruct((N_PAD, D), jnp.float32)),
)

_final_layer = pl.pallas_call(
    _final_body,
    out_shape=jax.ShapeDtypeStruct((N_PAD, D), jnp.float32),
)


def kernel(feats, edge_index, W1, b1, W2, b2):
    src = edge_index[0].astype(jnp.int32)
    dst = edge_index[1].astype(jnp.int32)
    pad = E_PAD - E_EDGES
    dummy = jnp.full((pad,), N_NODES, jnp.int32)
    srcp = jnp.concatenate([src, dummy]).reshape(NW, NCHUNKS, CHUNK)
    dstp = jnp.concatenate([dst, dummy]).reshape(NW, NCHUNKS, CHUNK)
    srcs = srcp.reshape(NSLABS, SLAB_CHUNKS, CHUNK)
    dsts = dstp.reshape(NSLABS, SLAB_CHUNKS, CHUNK)
    featsp = jnp.concatenate(
        [feats, jnp.zeros((N_PAD - N_NODES, D), jnp.float32)], axis=0)
    zero_rows = jnp.zeros((N_PAD, D), jnp.float32)
    lane = jnp.arange(D, dtype=jnp.int32)[None, :]
    ones_s = jnp.broadcast_to((lane >= 64).astype(jnp.float32), (CHUNK, D))
    ones_d = jnp.broadcast_to((lane < 64).astype(jnp.float32), (CHUNK, D))

    degree_kernel = _get_degree_kernel()
    agg_kernel = _get_agg_kernel()
    xw1_raw = _matmul1(featsp, W1)
    degp = degree_kernel(srcp, dstp, ones_s, ones_d, zero_rows)
    xw1, norm = _norm_scale(degp, xw1_raw)
    agg1p = agg_kernel(xw1, srcs, dsts, zero_rows)
    h1, xw2 = _mid_layer(agg1p, norm, b1.reshape(1, D), W2)
    agg2p = agg_kernel(xw2, srcs, dsts, zero_rows)
    h2 = _final_layer(agg2p, norm, b2.reshape(1, D))

    h1o = h1[:N_NODES]
    h2o = h2[:N_NODES]
    return ((h1o, h2o), h2o)
